# baseline (speedup 1.0000x reference)
import functools

import jax
import jax.numpy as jnp
from jax import lax
from jax.experimental import pallas as pl
from jax.experimental.pallas import tpu as pltpu
from jax.experimental.pallas import tpu_sc as plsc

N_CELL = 50000
E_CELL = 800000
N_DRUG = 40000
E_DRUG = 160000
B = 1024
BN_EPS = 1e-5

NC = 2
NS = 16
NW = NC * NS
CHUNK = 128
GROUP = 8
PAD_ROWS = 256


def _mesh():
    return plsc.VectorSubcoreMesh(
        core_axis_name="c", subcore_axis_name="s", num_cores=NC,
        num_subcores=NS)


def _round_up(x, m):
    return (x + m - 1) // m * m


@functools.lru_cache(None)
def _sc_degree(n_chunks, nacc):
    cw = n_chunks // NW
    groups = cw // GROUP

    @functools.partial(
        pl.kernel,
        out_type=jax.ShapeDtypeStruct((NC, nacc), jnp.float32),
        mesh=_mesh(),
        compiler_params=pltpu.CompilerParams(use_tc_tiling_on_sc=False),
        scratch_types=dict(
            didx=pltpu.VMEM((GROUP, CHUNK), jnp.int32),
            ones=pltpu.VMEM((CHUNK,), jnp.float32),
            acc=pltpu.VMEM_SHARED((nacc,), jnp.float32),
            semz=pltpu.SemaphoreType.DMA,
            sems=pltpu.SemaphoreType.DMA,
        ),
    )
    def deg_kernel(dst2d, zeros1, out, didx, ones, acc, semz, sems):
        c = lax.axis_index("c")
        s = lax.axis_index("s")
        w = s * NC + c
        for i in range(CHUNK // 16):
            ones[pl.ds(i * 16, 16)] = jnp.ones((16,), jnp.float32)
        rows_pt = nacc // NS
        base = pl.multiple_of(s * rows_pt, 8)
        pltpu.async_copy(zeros1.at[pl.ds(base, rows_pt)],
                         acc.at[pl.ds(base, rows_pt)], semz).wait()
        plsc.subcore_barrier()

        def body(g, _):
            gbase = w * cw + g * GROUP
            pltpu.sync_copy(dst2d.at[pl.ds(gbase, GROUP)], didx)
            descs = [
                pltpu.async_copy(ones, acc.at[didx.at[j]], sems, add=True)
                for j in range(GROUP)
            ]
            for d in descs:
                d.wait()
            return _

        lax.fori_loop(0, groups, body, None)
        plsc.subcore_barrier()
        pltpu.async_copy(acc.at[pl.ds(base, rows_pt)],
                         out.at[c].at[pl.ds(base, rows_pt)], semz).wait()

    return deg_kernel


@functools.lru_cache(None)
def _sc_rowagg16(n_chunks, nacc):
    cw = n_chunks // NW
    groups = cw // GROUP
    D = 16

    @functools.partial(
        pl.kernel,
        out_type=jax.ShapeDtypeStruct((NC, nacc, D), jnp.float32),
        mesh=_mesh(),
        compiler_params=pltpu.CompilerParams(use_tc_tiling_on_sc=False),
        scratch_types=dict(
            sidx=pltpu.VMEM((GROUP, CHUNK), jnp.int32),
            didx=pltpu.VMEM((GROUP, CHUNK), jnp.int32),
            rows=pltpu.VMEM((GROUP, CHUNK, D), jnp.float32),
            acc=pltpu.VMEM_SHARED((nacc, D), jnp.float32),
            semz=pltpu.SemaphoreType.DMA,
            semg=pltpu.SemaphoreType.DMA,
            sems=pltpu.SemaphoreType.DMA,
        ),
    )
    def agg16_kernel(src2d, dst2d, vals, zeros2, out,
                     sidx, didx, rows, acc, semz, semg, sems):
        c = lax.axis_index("c")
        s = lax.axis_index("s")
        w = s * NC + c
        rows_pt = nacc // NS
        base = pl.multiple_of(s * rows_pt, 8)
        pltpu.async_copy(zeros2.at[pl.ds(base, rows_pt)],
                         acc.at[pl.ds(base, rows_pt)], semz).wait()
        plsc.subcore_barrier()

        def body(g, _):
            gbase = w * cw + g * GROUP
            pltpu.sync_copy(src2d.at[pl.ds(gbase, GROUP)], sidx)
            pltpu.sync_copy(dst2d.at[pl.ds(gbase, GROUP)], didx)
            gd = [
                pltpu.async_copy(vals.at[sidx.at[j]], rows.at[j], semg)
                for j in range(GROUP)
            ]
            sd = []
            for j in range(GROUP):
                gd[j].wait()
                sd.append(pltpu.async_copy(rows.at[j], acc.at[didx.at[j]],
                                           sems, add=True))
            for d in sd:
                d.wait()
            return _

        lax.fori_loop(0, groups, body, None)
        plsc.subcore_barrier()
        pltpu.async_copy(acc.at[pl.ds(base, rows_pt)],
                         out.at[c].at[pl.ds(base, rows_pt)], semz).wait()

    return agg16_kernel


@functools.lru_cache(None)
def _sc_rowagg16x8(n_chunks, nacc):
    cw = n_chunks // NS
    groups = cw // GROUP
    D = 16

    @functools.partial(
        pl.kernel,
        out_type=jax.ShapeDtypeStruct((8, nacc, D), jnp.float32),
        mesh=_mesh(),
        compiler_params=pltpu.CompilerParams(use_tc_tiling_on_sc=False),
        scratch_types=dict(
            sidx=pltpu.VMEM((GROUP, CHUNK), jnp.int32),
            didx=pltpu.VMEM((GROUP, CHUNK), jnp.int32),
            gidx=pltpu.VMEM((GROUP, CHUNK), jnp.int32),
            rows=pltpu.VMEM((GROUP, CHUNK, D), jnp.float32),
            acc=pltpu.VMEM_SHARED((nacc, D), jnp.float32),
            semz=pltpu.SemaphoreType.DMA,
            semg=pltpu.SemaphoreType.DMA,
            sems=pltpu.SemaphoreType.DMA,
        ),
    )
    def agg8_kernel(src2d, dst2d, vals, zeros2, out,
                    sidx, didx, gidx, rows, acc, semz, semg, sems):
        c = lax.axis_index("c")
        s = lax.axis_index("s")
        rows_pt = nacc // NS
        base = pl.multiple_of(s * rows_pt, 8)

        for step in range(4):
            e = c + 2 * step
            pltpu.async_copy(zeros2.at[pl.ds(base, rows_pt)],
                             acc.at[pl.ds(base, rows_pt)], semz).wait()
            plsc.subcore_barrier()

            def body(g, _):
                gbase = s * cw + g * GROUP
                pltpu.sync_copy(src2d.at[pl.ds(gbase, GROUP)], sidx)
                pltpu.sync_copy(dst2d.at[pl.ds(gbase, GROUP)], didx)
                for j in range(GROUP):
                    for i in range(CHUNK // 16):
                        t = sidx[j, pl.ds(i * 16, 16)]
                        gidx[j, pl.ds(i * 16, 16)] = t * 8 + e
                gd = [
                    pltpu.async_copy(vals.at[gidx.at[j]], rows.at[j], semg)
                    for j in range(GROUP)
                ]
                sd = []
                for j in range(GROUP):
                    gd[j].wait()
                    sd.append(pltpu.async_copy(rows.at[j],
                                               acc.at[didx.at[j]],
                                               sems, add=True))
                for d in sd:
                    d.wait()
                return _

            lax.fori_loop(0, groups, body, None)
            plsc.subcore_barrier()
            pltpu.async_copy(acc.at[pl.ds(base, rows_pt)],
                             out.at[e].at[pl.ds(base, rows_pt)], semz).wait()
            plsc.subcore_barrier()

    return agg8_kernel


_RB = 1024


def _dot(a, b):
    return jnp.dot(a, b, preferred_element_type=jnp.float32)


def _tc_dinv_xs(p0, p1, xpad):
    n = xpad.shape[0]

    def body(p0_r, p1_r, x_r, dinv_r, xs_r):
        deg = 1.0 + p0_r[...] + p1_r[...]
        dinv = 1.0 / jnp.sqrt(deg)
        dinv_r[...] = dinv
        xs_r[...] = dinv * x_r[...]

    grid = n // _RB
    return pl.pallas_call(
        body,
        grid=(grid,),
        in_specs=[
            pl.BlockSpec((_RB, 1), lambda i: (i, 0)),
            pl.BlockSpec((_RB, 1), lambda i: (i, 0)),
            pl.BlockSpec((_RB, 16), lambda i: (i, 0)),
        ],
        out_specs=[
            pl.BlockSpec((_RB, 1), lambda i: (i, 0)),
            pl.BlockSpec((_RB, 16), lambda i: (i, 0)),
        ],
        out_shape=[
            jax.ShapeDtypeStruct((n, 1), jnp.float32),
            jax.ShapeDtypeStruct((n, 16), jnp.float32),
        ],
    )(p0, p1, xpad)


def _tc_cell_dense1(p0, p1, xs, dinv, W1p, b1, W2):
    n = xs.shape[0]

    def body(p0_r, p1_r, xs_r, dinv_r, w1_r, b1_r, w2_r, v_r):
        dinv = dinv_r[...]
        u = dinv * (p0_r[...] + p1_r[...] + xs_r[...])
        h1 = jax.nn.relu(_dot(u, w1_r[...]) + b1_r[...])
        v_r[...] = dinv * _dot(h1, w2_r[...])

    grid = n // _RB
    return pl.pallas_call(
        body,
        grid=(grid,),
        in_specs=[
            pl.BlockSpec((_RB, 16), lambda i: (i, 0)),
            pl.BlockSpec((_RB, 16), lambda i: (i, 0)),
            pl.BlockSpec((_RB, 16), lambda i: (i, 0)),
            pl.BlockSpec((_RB, 1), lambda i: (i, 0)),
            pl.BlockSpec((16, 256), lambda i: (0, 0)),
            pl.BlockSpec((1, 256), lambda i: (0, 0)),
            pl.BlockSpec((256, 128), lambda i: (0, 0)),
        ],
        out_specs=pl.BlockSpec((_RB, 128), lambda i: (i, 0)),
        out_shape=jax.ShapeDtypeStruct((n, 128), jnp.float32),
    )(p0, p1, xs, dinv, W1p, b1, W2)


def _segmax_update(ids_r, h, out_r):
    ids = ids_r[...]
    lo = ids[0, 0]
    hi = ids[_RB - 1, 0]
    neg = jnp.float32(-jnp.inf)

    def seg_body(j, _):
        seg = lo + j
        m = jnp.where(ids == seg, h, neg)
        red = jnp.max(m, axis=0, keepdims=True)
        cur = out_r[pl.ds(seg, 1), :]
        out_r[pl.ds(seg, 1), :] = jnp.maximum(cur, red)
        return _

    lax.fori_loop(0, hi - lo + 1, seg_body, None)


def _tc_cell_pool(qs, v, dinv, b2, ids):
    q0, q1, q2, q3, q4, q5, q6, q7 = qs
    n = v.shape[0]

    def body(q0_r, q1_r, q2_r, q3_r, q4_r, q5_r, q6_r, q7_r, v_r, dinv_r,
             b2_r, ids_r, out_r):
        i = pl.program_id(0)

        @pl.when(i == 0)
        def _init():
            out_r[...] = jnp.full((B + 8, 128), -jnp.inf, jnp.float32)

        qcat = jnp.concatenate(
            [q0_r[...], q1_r[...], q2_r[...], q3_r[...],
             q4_r[...], q5_r[...], q6_r[...], q7_r[...]], axis=1)
        h2 = dinv_r[...] * (qcat + v_r[...]) + b2_r[...]
        _segmax_update(ids_r, h2, out_r)

    grid = n // _RB
    return pl.pallas_call(
        body,
        grid=(grid,),
        in_specs=[pl.BlockSpec((_RB, 16), lambda i: (i, 0))] * 8 + [
            pl.BlockSpec((_RB, 128), lambda i: (i, 0)),
            pl.BlockSpec((_RB, 1), lambda i: (i, 0)),
            pl.BlockSpec((1, 128), lambda i: (0, 0)),
            pl.BlockSpec((_RB, 1), lambda i: (i, 0)),
        ],
        out_specs=pl.BlockSpec((B + 8, 128), lambda i: (0, 0)),
        out_shape=jax.ShapeDtypeStruct((B + 8, 128), jnp.float32),
    )(q0, q1, q2, q3, q4, q5, q6, q7, v, dinv, b2, ids)


def _tc_drug_dense1(p0, p1, xpad, W1p, b1, s1, o1, W2, b2, s2, o2):
    n = xpad.shape[0]

    def body(p0_r, p1_r, x_r, w1_r, b1_r, s1_r, o1_r, w2_r, b2_r, s2_r,
             o2_r, out_r):
        a = p0_r[...] + p1_r[...] + x_r[...]
        t = _dot(a, w1_r[...]) + b1_r[...]
        t = t * s1_r[...] + o1_r[...]
        t = jax.nn.relu(t)
        t = _dot(t, w2_r[...]) + b2_r[...]
        t = jax.nn.relu(t)
        out_r[...] = t * s2_r[...] + o2_r[...]

    grid = n // _RB
    return pl.pallas_call(
        body,
        grid=(grid,),
        in_specs=[
            pl.BlockSpec((_RB, 16), lambda i: (i, 0)),
            pl.BlockSpec((_RB, 16), lambda i: (i, 0)),
            pl.BlockSpec((_RB, 16), lambda i: (i, 0)),
            pl.BlockSpec((16, 128), lambda i: (0, 0)),
            pl.BlockSpec((1, 128), lambda i: (0, 0)),
            pl.BlockSpec((1, 128), lambda i: (0, 0)),
            pl.BlockSpec((1, 128), lambda i: (0, 0)),
            pl.BlockSpec((128, 128), lambda i: (0, 0)),
            pl.BlockSpec((1, 128), lambda i: (0, 0)),
            pl.BlockSpec((1, 128), lambda i: (0, 0)),
            pl.BlockSpec((1, 128), lambda i: (0, 0)),
        ],
        out_specs=pl.BlockSpec((_RB, 128), lambda i: (i, 0)),
        out_shape=jax.ShapeDtypeStruct((n, 128), jnp.float32),
    )(p0, p1, xpad, W1p, b1, s1, o1, W2, b2, s2, o2)


def _tc_drug_pool(qs, hd1, W1, b1, s1, o1, W2, b2, s2, o2, ids):
    q0, q1, q2, q3, q4, q5, q6, q7 = qs
    n = hd1.shape[0]

    def body(q0_r, q1_r, q2_r, q3_r, q4_r, q5_r, q6_r, q7_r, h_r, w1_r,
             b1_r, s1_r, o1_r, w2_r, b2_r, s2_r, o2_r, ids_r, out_r):
        i = pl.program_id(0)

        @pl.when(i == 0)
        def _init():
            out_r[...] = jnp.full((B + 8, 128), -jnp.inf, jnp.float32)

        a = jnp.concatenate(
            [q0_r[...], q1_r[...], q2_r[...], q3_r[...],
             q4_r[...], q5_r[...], q6_r[...], q7_r[...]],
            axis=1) + h_r[...]
        t = _dot(a, w1_r[...]) + b1_r[...]
        t = t * s1_r[...] + o1_r[...]
        t = jax.nn.relu(t)
        t = _dot(t, w2_r[...]) + b2_r[...]
        t = jax.nn.relu(t)
        h2 = t * s2_r[...] + o2_r[...]
        _segmax_update(ids_r, h2, out_r)

    grid = n // _RB
    return pl.pallas_call(
        body,
        grid=(grid,),
        in_specs=[pl.BlockSpec((_RB, 16), lambda i: (i, 0))] * 8 + [
            pl.BlockSpec((_RB, 128), lambda i: (i, 0)),
            pl.BlockSpec((128, 128), lambda i: (0, 0)),
            pl.BlockSpec((1, 128), lambda i: (0, 0)),
            pl.BlockSpec((1, 128), lambda i: (0, 0)),
            pl.BlockSpec((1, 128), lambda i: (0, 0)),
            pl.BlockSpec((128, 128), lambda i: (0, 0)),
            pl.BlockSpec((1, 128), lambda i: (0, 0)),
            pl.BlockSpec((1, 128), lambda i: (0, 0)),
            pl.BlockSpec((1, 128), lambda i: (0, 0)),
            pl.BlockSpec((_RB, 1), lambda i: (i, 0)),
        ],
        out_specs=pl.BlockSpec((B + 8, 128), lambda i: (0, 0)),
        out_shape=jax.ShapeDtypeStruct((B + 8, 128), jnp.float32),
    )(q0, q1, q2, q3, q4, q5, q6, q7, hd1, W1, b1, s1, o1, W2, b2, s2, o2,
      ids)


def _tc_head(gc, gd, wts):
    (cL1, cL1b, cBNs, cBNo, cL2, cL2b,
     dL1, dL1b, dBs, dBo, dL2, dL2b,
     fW1, fb1, fB1s, fB1o, fW2, fb2, fB2s, fB2o, fW3, fb3) = wts

    def body(gc_r, gd_r, cL1_r, cL1b_r, cBNs_r, cBNo_r, cL2_r, cL2b_r,
             dL1_r, dL1b_r, dBs_r, dBo_r, dL2_r, dL2b_r,
             fW1_r, fb1_r, fB1s_r, fB1o_r, fW2_r, fb2_r, fB2s_r, fB2o_r,
             fW3_r, fb3_r, out_r):
        g = _dot(gc_r[...], cL1_r[...]) + cL1b_r[...]
        g = g * cBNs_r[...] + cBNo_r[...]
        g = jax.nn.relu(g)
        g = _dot(g, cL2_r[...]) + cL2b_r[...]
        cell_emb = jax.nn.relu(g)

        g = _dot(gd_r[...], dL1_r[...]) + dL1b_r[...]
        g = g * dBs_r[...] + dBo_r[...]
        g = jax.nn.relu(g)
        g = _dot(g, dL2_r[...]) + dL2b_r[...]
        drug_emb = jax.nn.relu(g)

        z = jnp.concatenate([cell_emb, drug_emb], axis=1)
        z = _dot(z, fW1_r[...]) + fb1_r[...]
        z = z * fB1s_r[...] + fB1o_r[...]
        z = jnp.where(z > 0, z, jnp.exp(z) - 1.0)
        z = _dot(z, fW2_r[...]) + fb2_r[...]
        z = z * fB2s_r[...] + fB2o_r[...]
        z = jnp.where(z > 0, z, jnp.exp(z) - 1.0)
        out_r[...] = _dot(z, fW3_r[...]) + fb3_r[...]

    def _spec(a):
        zero = (0,) * a.ndim
        return pl.BlockSpec(a.shape, lambda i, z=zero: z)

    specs = [_spec(a)
             for a in (gc, gd, cL1, cL1b, cBNs, cBNo, cL2, cL2b,
                       dL1, dL1b, dBs, dBo, dL2, dL2b,
                       fW1, fb1, fB1s, fB1o, fW2, fb2, fB2s, fB2o, fW3, fb3)]
    return pl.pallas_call(
        body,
        grid=(1,),
        in_specs=specs,
        out_specs=pl.BlockSpec((B, 1), lambda i: (0, 0)),
        out_shape=jax.ShapeDtypeStruct((B, 1), jnp.float32),
    )(gc, gd, cL1, cL1b, cBNs, cBNo, cL2, cL2b,
      dL1, dL1b, dBs, dBo, dL2, dL2b,
      fW1, fb1, fB1s, fB1o, fW2, fb2, fB2s, fB2o, fW3, fb3)


def _prep_edges(ei, n_nodes, e_pad):
    src = ei[0]
    dst = ei[1]
    e = src.shape[0]
    pad_n = e_pad - e
    src_p = jnp.concatenate([src, jnp.zeros((pad_n,), jnp.int32)])
    dst_p = jnp.concatenate(
        [dst, n_nodes + (jnp.arange(pad_n, dtype=jnp.int32) % PAD_ROWS)])
    return src_p.reshape(e_pad // CHUNK, CHUNK), dst_p.reshape(
        e_pad // CHUNK, CHUNK)


def _padrows(x, n):
    return jnp.concatenate(
        [x, jnp.zeros((n - x.shape[0], x.shape[1]), x.dtype)], axis=0)


def _padids(ids, n):
    pad = jnp.full((n - ids.shape[0],), B, jnp.int32)
    return jnp.concatenate([ids, pad]).reshape(n, 1)


def _pad16(x):
    n, d = x.shape
    return jnp.concatenate([x, jnp.zeros((n, 16 - d), jnp.float32)], axis=1)


def _bn_fold(g, b):
    scale = g / jnp.sqrt(1.0 + BN_EPS)
    return scale.reshape(1, -1), b.reshape(1, -1)


def _row(v):
    return v.reshape(1, -1)


def kernel(cell_x, cell_edge_index, cell_batch, drug_x, drug_edge_index,
           drug_batch, params):
    p = params
    ep_cell = _round_up(E_CELL, CHUNK * NW * GROUP)
    ep_drug = _round_up(E_DRUG, CHUNK * NW * GROUP)
    nacc_c = _round_up(N_CELL + PAD_ROWS, NS * 128)
    nacc_d = _round_up(N_DRUG + PAD_ROWS, NS * 128)
    csrc2, cdst2 = _prep_edges(cell_edge_index, N_CELL, ep_cell)
    dsrc2, ddst2 = _prep_edges(drug_edge_index, N_DRUG, ep_drug)
    cx16 = _padrows(_pad16(cell_x), nacc_c)
    dx16 = _padrows(_pad16(drug_x), nacc_d)
    cids = _padids(cell_batch, nacc_c)
    dids = _padids(drug_batch, nacc_d)
    zc1 = jnp.zeros((nacc_c,), jnp.float32)
    zc16 = jnp.zeros((nacc_c, 16), jnp.float32)
    zd16 = jnp.zeros((nacc_d, 16), jnp.float32)

    nch_c = ep_cell // CHUNK
    deg = _sc_degree(nch_c, nacc_c)(cdst2, zc1)
    deg0 = deg[0].reshape(nacc_c, 1)
    deg1 = deg[1].reshape(nacc_c, 1)
    dinv, xs = _tc_dinv_xs(deg0, deg1, cx16)

    agg1 = _sc_rowagg16(nch_c, nacc_c)(csrc2, cdst2, xs, zc16)

    W1p = jnp.zeros((16, 256), jnp.float32).at[:4].set(p['cW1'])
    v = _tc_cell_dense1(agg1[0], agg1[1], xs, dinv, W1p, _row(p['cb1']),
                        p['cW2'])

    vq = v.reshape(nacc_c * 8, 16)
    agg2 = _sc_rowagg16x8(nch_c, nacc_c)(csrc2, cdst2, vq, zc16)
    g_cell = _tc_cell_pool(
        tuple(agg2[e] for e in range(8)), v, dinv, _row(p['cb2']),
        cids)[:B]

    nch_d = ep_drug // CHUNK
    dagg1 = _sc_rowagg16(nch_d, nacc_d)(dsrc2, ddst2, dx16, zd16)
    g1W1p = jnp.zeros((16, 128), jnp.float32).at[:9].set(p['g1W1'])
    s1, o1 = _bn_fold(p['g1Mg'], p['g1Mb'])
    s2, o2 = _bn_fold(p['d1Bg'], p['d1Bb'])
    hd1 = _tc_drug_dense1(
        dagg1[0], dagg1[1], dx16, g1W1p,
        _row(p['g1b1']), s1, o1, p['g1W2'], _row(p['g1b2']), s2, o2)

    hq = hd1.reshape(nacc_d * 8, 16)
    dagg2 = _sc_rowagg16x8(nch_d, nacc_d)(dsrc2, ddst2, hq, zd16)
    s3, o3 = _bn_fold(p['g2Mg'], p['g2Mb'])
    s4, o4 = _bn_fold(p['d2Bg'], p['d2Bb'])
    g_drug = _tc_drug_pool(
        tuple(dagg2[e] for e in range(8)), hd1, p['g2W1'],
        _row(p['g2b1']), s3, o3,
        p['g2W2'], _row(p['g2b2']), s4, o4, dids)[:B]

    cs, co = _bn_fold(p['cBNg'], p['cBNb'])
    ds_, do = _bn_fold(p['dBg'], p['dBb'])
    f1s, f1o = _bn_fold(p['fB1g'], p['fB1b'])
    f2s, f2o = _bn_fold(p['fB2g'], p['fB2b'])
    out = _tc_head(g_cell, g_drug, (
        p['cL1'], _row(p['cL1b']), cs, co, p['cL2'], _row(p['cL2b']),
        p['dL1'], _row(p['dL1b']), ds_, do, p['dL2'], _row(p['dL2b']),
        p['fW1'], _row(p['fb1']), f1s, f1o,
        p['fW2'], _row(p['fb2']), f2s, f2o,
        p['fW3'], _row(p['fb3'])))
    return out.reshape(B)


# speedup vs baseline: 1.4206x; 1.4206x over previous
import functools

import jax
import jax.numpy as jnp
from jax import lax
from jax.experimental import pallas as pl
from jax.experimental.pallas import tpu as pltpu
from jax.experimental.pallas import tpu_sc as plsc

N_CELL = 50000
E_CELL = 800000
N_DRUG = 40000
E_DRUG = 160000
B = 1024
BN_EPS = 1e-5

NC = 2
NS = 16
NW = NC * NS
CHUNK = 128
GROUP = 8
PAD_ROWS = 256


def _mesh():
    return plsc.VectorSubcoreMesh(
        core_axis_name="c", subcore_axis_name="s", num_cores=NC,
        num_subcores=NS)


def _round_up(x, m):
    return (x + m - 1) // m * m


@functools.lru_cache(None)
def _sc_degree(n_chunks, nacc):
    cw = n_chunks // NW
    groups = cw // GROUP

    @functools.partial(
        pl.kernel,
        out_type=jax.ShapeDtypeStruct((NC, nacc), jnp.float32),
        mesh=_mesh(),
        compiler_params=pltpu.CompilerParams(use_tc_tiling_on_sc=False),
        scratch_types=dict(
            didx=pltpu.VMEM((GROUP, CHUNK), jnp.int32),
            ones=pltpu.VMEM((CHUNK,), jnp.float32),
            acc=pltpu.VMEM_SHARED((nacc,), jnp.float32),
            semz=pltpu.SemaphoreType.DMA,
            sems=pltpu.SemaphoreType.DMA,
        ),
    )
    def deg_kernel(dst2d, zeros1, out, didx, ones, acc, semz, sems):
        c = lax.axis_index("c")
        s = lax.axis_index("s")
        w = s * NC + c
        for i in range(CHUNK // 16):
            ones[pl.ds(i * 16, 16)] = jnp.ones((16,), jnp.float32)
        rows_pt = nacc // NS
        base = pl.multiple_of(s * rows_pt, 8)
        pltpu.async_copy(zeros1.at[pl.ds(base, rows_pt)],
                         acc.at[pl.ds(base, rows_pt)], semz).wait()
        plsc.subcore_barrier()

        def body(g, _):
            gbase = w * cw + g * GROUP
            pltpu.sync_copy(dst2d.at[pl.ds(gbase, GROUP)], didx)
            descs = [
                pltpu.async_copy(ones, acc.at[didx.at[j]], sems, add=True)
                for j in range(GROUP)
            ]
            for d in descs:
                d.wait()
            return _

        lax.fori_loop(0, groups, body, None)
        plsc.subcore_barrier()
        pltpu.async_copy(acc.at[pl.ds(base, rows_pt)],
                         out.at[c].at[pl.ds(base, rows_pt)], semz).wait()

    return deg_kernel


@functools.lru_cache(None)
def _sc_rowagg16(n_chunks, nacc):
    cw = n_chunks // NW
    groups = cw // GROUP
    D = 16

    @functools.partial(
        pl.kernel,
        out_type=jax.ShapeDtypeStruct((NC, nacc, D), jnp.float32),
        mesh=_mesh(),
        compiler_params=pltpu.CompilerParams(use_tc_tiling_on_sc=False),
        scratch_types=dict(
            sidx=pltpu.VMEM((GROUP, CHUNK), jnp.int32),
            didx=pltpu.VMEM((GROUP, CHUNK), jnp.int32),
            rows=pltpu.VMEM((GROUP, CHUNK, D), jnp.float32),
            acc=pltpu.VMEM_SHARED((nacc, D), jnp.float32),
            vstage=pltpu.VMEM_SHARED((nacc, D), jnp.float32),
            semz=pltpu.SemaphoreType.DMA,
            semg=pltpu.SemaphoreType.DMA,
            sems=pltpu.SemaphoreType.DMA,
        ),
    )
    def agg16_kernel(src2d, dst2d, vals, zeros2, out,
                     sidx, didx, rows, acc, vstage, semz, semg, sems):
        c = lax.axis_index("c")
        s = lax.axis_index("s")
        w = s * NC + c
        rows_pt = nacc // NS
        base = pl.multiple_of(s * rows_pt, 8)
        pltpu.async_copy(zeros2.at[pl.ds(base, rows_pt)],
                         acc.at[pl.ds(base, rows_pt)], semz).wait()
        pltpu.async_copy(vals.at[pl.ds(base, rows_pt)],
                         vstage.at[pl.ds(base, rows_pt)], semz).wait()
        plsc.subcore_barrier()

        def body(g, _):
            gbase = w * cw + g * GROUP
            pltpu.sync_copy(src2d.at[pl.ds(gbase, GROUP)], sidx)
            pltpu.sync_copy(dst2d.at[pl.ds(gbase, GROUP)], didx)
            gd = [
                pltpu.async_copy(vstage.at[sidx.at[j]], rows.at[j], semg)
                for j in range(GROUP)
            ]
            sd = []
            for j in range(GROUP):
                gd[j].wait()
                sd.append(pltpu.async_copy(rows.at[j], acc.at[didx.at[j]],
                                           sems, add=True))
            for d in sd:
                d.wait()
            return _

        lax.fori_loop(0, groups, body, None)
        plsc.subcore_barrier()
        pltpu.async_copy(acc.at[pl.ds(base, rows_pt)],
                         out.at[c].at[pl.ds(base, rows_pt)], semz).wait()

    return agg16_kernel


@functools.lru_cache(None)
def _sc_rowagg16x8(n_chunks, nacc):
    cw = n_chunks // NS
    groups = cw // GROUP
    D = 16

    @functools.partial(
        pl.kernel,
        out_type=jax.ShapeDtypeStruct((8, nacc, D), jnp.float32),
        mesh=_mesh(),
        compiler_params=pltpu.CompilerParams(use_tc_tiling_on_sc=False),
        scratch_types=dict(
            sidx=pltpu.VMEM((GROUP, CHUNK), jnp.int32),
            didx=pltpu.VMEM((GROUP, CHUNK), jnp.int32),
            rows=pltpu.VMEM((GROUP, CHUNK, D), jnp.float32),
            acc=pltpu.VMEM_SHARED((nacc, D), jnp.float32),
            vstage=pltpu.VMEM_SHARED((nacc, D), jnp.float32),
            semz=pltpu.SemaphoreType.DMA,
            semg=pltpu.SemaphoreType.DMA,
            sems=pltpu.SemaphoreType.DMA,
        ),
    )
    def agg8_kernel(src2d, dst2d, vsplit, zeros2, out,
                    sidx, didx, rows, acc, vstage, semz, semg, sems):
        c = lax.axis_index("c")
        s = lax.axis_index("s")
        rows_pt = nacc // NS
        base = pl.multiple_of(s * rows_pt, 8)

        for step in range(4):
            e = c + 2 * step
            pltpu.async_copy(zeros2.at[pl.ds(base, rows_pt)],
                             acc.at[pl.ds(base, rows_pt)], semz).wait()
            pltpu.async_copy(vsplit.at[e].at[pl.ds(base, rows_pt)],
                             vstage.at[pl.ds(base, rows_pt)], semz).wait()
            plsc.subcore_barrier()

            def body(g, _):
                gbase = s * cw + g * GROUP
                pltpu.sync_copy(src2d.at[pl.ds(gbase, GROUP)], sidx)
                pltpu.sync_copy(dst2d.at[pl.ds(gbase, GROUP)], didx)
                gd = [
                    pltpu.async_copy(vstage.at[sidx.at[j]], rows.at[j],
                                     semg)
                    for j in range(GROUP)
                ]
                sd = []
                for j in range(GROUP):
                    gd[j].wait()
                    sd.append(pltpu.async_copy(rows.at[j],
                                               acc.at[didx.at[j]],
                                               sems, add=True))
                for d in sd:
                    d.wait()
                return _

            lax.fori_loop(0, groups, body, None)
            plsc.subcore_barrier()
            pltpu.async_copy(acc.at[pl.ds(base, rows_pt)],
                             out.at[e].at[pl.ds(base, rows_pt)], semz).wait()
            plsc.subcore_barrier()

    return agg8_kernel


_RB = 1024


def _dot(a, b):
    return jnp.dot(a, b, preferred_element_type=jnp.float32)


def _tc_dinv_xs(p0, p1, xpad):
    n = xpad.shape[0]

    def body(p0_r, p1_r, x_r, dinv_r, xs_r):
        deg = 1.0 + p0_r[...] + p1_r[...]
        dinv = 1.0 / jnp.sqrt(deg)
        dinv_r[...] = dinv
        xs_r[...] = dinv * x_r[...]

    grid = n // _RB
    return pl.pallas_call(
        body,
        grid=(grid,),
        in_specs=[
            pl.BlockSpec((_RB, 1), lambda i: (i, 0)),
            pl.BlockSpec((_RB, 1), lambda i: (i, 0)),
            pl.BlockSpec((_RB, 16), lambda i: (i, 0)),
        ],
        out_specs=[
            pl.BlockSpec((_RB, 1), lambda i: (i, 0)),
            pl.BlockSpec((_RB, 16), lambda i: (i, 0)),
        ],
        out_shape=[
            jax.ShapeDtypeStruct((n, 1), jnp.float32),
            jax.ShapeDtypeStruct((n, 16), jnp.float32),
        ],
    )(p0, p1, xpad)


def _tc_cell_dense1(p0, p1, xs, dinv, W1p, b1, W2):
    n = xs.shape[0]

    def body(p0_r, p1_r, xs_r, dinv_r, w1_r, b1_r, w2_r, v_r):
        dinv = dinv_r[...]
        u = dinv * (p0_r[...] + p1_r[...] + xs_r[...])
        h1 = jax.nn.relu(_dot(u, w1_r[...]) + b1_r[...])
        v_r[...] = dinv * _dot(h1, w2_r[...])

    grid = n // _RB
    return pl.pallas_call(
        body,
        grid=(grid,),
        in_specs=[
            pl.BlockSpec((_RB, 16), lambda i: (i, 0)),
            pl.BlockSpec((_RB, 16), lambda i: (i, 0)),
            pl.BlockSpec((_RB, 16), lambda i: (i, 0)),
            pl.BlockSpec((_RB, 1), lambda i: (i, 0)),
            pl.BlockSpec((16, 256), lambda i: (0, 0)),
            pl.BlockSpec((1, 256), lambda i: (0, 0)),
            pl.BlockSpec((256, 128), lambda i: (0, 0)),
        ],
        out_specs=pl.BlockSpec((_RB, 128), lambda i: (i, 0)),
        out_shape=jax.ShapeDtypeStruct((n, 128), jnp.float32),
    )(p0, p1, xs, dinv, W1p, b1, W2)


def _segmax_update(ids_r, h, out_r):
    ids = ids_r[...]
    lo = ids[0, 0]
    hi = ids[_RB - 1, 0]
    neg = jnp.float32(-jnp.inf)

    def seg_body(j, _):
        seg = lo + j
        m = jnp.where(ids == seg, h, neg)
        red = jnp.max(m, axis=0, keepdims=True)
        cur = out_r[pl.ds(seg, 1), :]
        out_r[pl.ds(seg, 1), :] = jnp.maximum(cur, red)
        return _

    lax.fori_loop(0, hi - lo + 1, seg_body, None)


def _tc_cell_pool(qs, v, dinv, b2, ids):
    q0, q1, q2, q3, q4, q5, q6, q7 = qs
    n = v.shape[0]

    def body(q0_r, q1_r, q2_r, q3_r, q4_r, q5_r, q6_r, q7_r, v_r, dinv_r,
             b2_r, ids_r, out_r):
        i = pl.program_id(0)

        @pl.when(i == 0)
        def _init():
            out_r[...] = jnp.full((B + 8, 128), -jnp.inf, jnp.float32)

        qcat = jnp.concatenate(
            [q0_r[...], q1_r[...], q2_r[...], q3_r[...],
             q4_r[...], q5_r[...], q6_r[...], q7_r[...]], axis=1)
        h2 = dinv_r[...] * (qcat + v_r[...]) + b2_r[...]
        _segmax_update(ids_r, h2, out_r)

    grid = n // _RB
    return pl.pallas_call(
        body,
        grid=(grid,),
        in_specs=[pl.BlockSpec((_RB, 16), lambda i: (i, 0))] * 8 + [
            pl.BlockSpec((_RB, 128), lambda i: (i, 0)),
            pl.BlockSpec((_RB, 1), lambda i: (i, 0)),
            pl.BlockSpec((1, 128), lambda i: (0, 0)),
            pl.BlockSpec((_RB, 1), lambda i: (i, 0)),
        ],
        out_specs=pl.BlockSpec((B + 8, 128), lambda i: (0, 0)),
        out_shape=jax.ShapeDtypeStruct((B + 8, 128), jnp.float32),
    )(q0, q1, q2, q3, q4, q5, q6, q7, v, dinv, b2, ids)


def _tc_drug_dense1(p0, p1, xpad, W1p, b1, s1, o1, W2, b2, s2, o2):
    n = xpad.shape[0]

    def body(p0_r, p1_r, x_r, w1_r, b1_r, s1_r, o1_r, w2_r, b2_r, s2_r,
             o2_r, out_r):
        a = p0_r[...] + p1_r[...] + x_r[...]
        t = _dot(a, w1_r[...]) + b1_r[...]
        t = t * s1_r[...] + o1_r[...]
        t = jax.nn.relu(t)
        t = _dot(t, w2_r[...]) + b2_r[...]
        t = jax.nn.relu(t)
        out_r[...] = t * s2_r[...] + o2_r[...]

    grid = n // _RB
    return pl.pallas_call(
        body,
        grid=(grid,),
        in_specs=[
            pl.BlockSpec((_RB, 16), lambda i: (i, 0)),
            pl.BlockSpec((_RB, 16), lambda i: (i, 0)),
            pl.BlockSpec((_RB, 16), lambda i: (i, 0)),
            pl.BlockSpec((16, 128), lambda i: (0, 0)),
            pl.BlockSpec((1, 128), lambda i: (0, 0)),
            pl.BlockSpec((1, 128), lambda i: (0, 0)),
            pl.BlockSpec((1, 128), lambda i: (0, 0)),
            pl.BlockSpec((128, 128), lambda i: (0, 0)),
            pl.BlockSpec((1, 128), lambda i: (0, 0)),
            pl.BlockSpec((1, 128), lambda i: (0, 0)),
            pl.BlockSpec((1, 128), lambda i: (0, 0)),
        ],
        out_specs=pl.BlockSpec((_RB, 128), lambda i: (i, 0)),
        out_shape=jax.ShapeDtypeStruct((n, 128), jnp.float32),
    )(p0, p1, xpad, W1p, b1, s1, o1, W2, b2, s2, o2)


def _tc_drug_pool(qs, hd1, W1, b1, s1, o1, W2, b2, s2, o2, ids):
    q0, q1, q2, q3, q4, q5, q6, q7 = qs
    n = hd1.shape[0]

    def body(q0_r, q1_r, q2_r, q3_r, q4_r, q5_r, q6_r, q7_r, h_r, w1_r,
             b1_r, s1_r, o1_r, w2_r, b2_r, s2_r, o2_r, ids_r, out_r):
        i = pl.program_id(0)

        @pl.when(i == 0)
        def _init():
            out_r[...] = jnp.full((B + 8, 128), -jnp.inf, jnp.float32)

        a = jnp.concatenate(
            [q0_r[...], q1_r[...], q2_r[...], q3_r[...],
             q4_r[...], q5_r[...], q6_r[...], q7_r[...]],
            axis=1) + h_r[...]
        t = _dot(a, w1_r[...]) + b1_r[...]
        t = t * s1_r[...] + o1_r[...]
        t = jax.nn.relu(t)
        t = _dot(t, w2_r[...]) + b2_r[...]
        t = jax.nn.relu(t)
        h2 = t * s2_r[...] + o2_r[...]
        _segmax_update(ids_r, h2, out_r)

    grid = n // _RB
    return pl.pallas_call(
        body,
        grid=(grid,),
        in_specs=[pl.BlockSpec((_RB, 16), lambda i: (i, 0))] * 8 + [
            pl.BlockSpec((_RB, 128), lambda i: (i, 0)),
            pl.BlockSpec((128, 128), lambda i: (0, 0)),
            pl.BlockSpec((1, 128), lambda i: (0, 0)),
            pl.BlockSpec((1, 128), lambda i: (0, 0)),
            pl.BlockSpec((1, 128), lambda i: (0, 0)),
            pl.BlockSpec((128, 128), lambda i: (0, 0)),
            pl.BlockSpec((1, 128), lambda i: (0, 0)),
            pl.BlockSpec((1, 128), lambda i: (0, 0)),
            pl.BlockSpec((1, 128), lambda i: (0, 0)),
            pl.BlockSpec((_RB, 1), lambda i: (i, 0)),
        ],
        out_specs=pl.BlockSpec((B + 8, 128), lambda i: (0, 0)),
        out_shape=jax.ShapeDtypeStruct((B + 8, 128), jnp.float32),
    )(q0, q1, q2, q3, q4, q5, q6, q7, hd1, W1, b1, s1, o1, W2, b2, s2, o2,
      ids)


def _tc_head(gc, gd, wts):
    (cL1, cL1b, cBNs, cBNo, cL2, cL2b,
     dL1, dL1b, dBs, dBo, dL2, dL2b,
     fW1, fb1, fB1s, fB1o, fW2, fb2, fB2s, fB2o, fW3, fb3) = wts

    def body(gc_r, gd_r, cL1_r, cL1b_r, cBNs_r, cBNo_r, cL2_r, cL2b_r,
             dL1_r, dL1b_r, dBs_r, dBo_r, dL2_r, dL2b_r,
             fW1_r, fb1_r, fB1s_r, fB1o_r, fW2_r, fb2_r, fB2s_r, fB2o_r,
             fW3_r, fb3_r, out_r):
        g = _dot(gc_r[...], cL1_r[...]) + cL1b_r[...]
        g = g * cBNs_r[...] + cBNo_r[...]
        g = jax.nn.relu(g)
        g = _dot(g, cL2_r[...]) + cL2b_r[...]
        cell_emb = jax.nn.relu(g)

        g = _dot(gd_r[...], dL1_r[...]) + dL1b_r[...]
        g = g * dBs_r[...] + dBo_r[...]
        g = jax.nn.relu(g)
        g = _dot(g, dL2_r[...]) + dL2b_r[...]
        drug_emb = jax.nn.relu(g)

        z = jnp.concatenate([cell_emb, drug_emb], axis=1)
        z = _dot(z, fW1_r[...]) + fb1_r[...]
        z = z * fB1s_r[...] + fB1o_r[...]
        z = jnp.where(z > 0, z, jnp.exp(z) - 1.0)
        z = _dot(z, fW2_r[...]) + fb2_r[...]
        z = z * fB2s_r[...] + fB2o_r[...]
        z = jnp.where(z > 0, z, jnp.exp(z) - 1.0)
        out_r[...] = _dot(z, fW3_r[...]) + fb3_r[...]

    def _spec(a):
        zero = (0,) * a.ndim
        return pl.BlockSpec(a.shape, lambda i, z=zero: z)

    specs = [_spec(a)
             for a in (gc, gd, cL1, cL1b, cBNs, cBNo, cL2, cL2b,
                       dL1, dL1b, dBs, dBo, dL2, dL2b,
                       fW1, fb1, fB1s, fB1o, fW2, fb2, fB2s, fB2o, fW3, fb3)]
    return pl.pallas_call(
        body,
        grid=(1,),
        in_specs=specs,
        out_specs=pl.BlockSpec((B, 1), lambda i: (0, 0)),
        out_shape=jax.ShapeDtypeStruct((B, 1), jnp.float32),
    )(gc, gd, cL1, cL1b, cBNs, cBNo, cL2, cL2b,
      dL1, dL1b, dBs, dBo, dL2, dL2b,
      fW1, fb1, fB1s, fB1o, fW2, fb2, fB2s, fB2o, fW3, fb3)


def _prep_edges(ei, n_nodes, e_pad):
    src = ei[0]
    dst = ei[1]
    e = src.shape[0]
    pad_n = e_pad - e
    src_p = jnp.concatenate([src, jnp.zeros((pad_n,), jnp.int32)])
    dst_p = jnp.concatenate(
        [dst, n_nodes + (jnp.arange(pad_n, dtype=jnp.int32) % PAD_ROWS)])
    return src_p.reshape(e_pad // CHUNK, CHUNK), dst_p.reshape(
        e_pad // CHUNK, CHUNK)


def _padrows(x, n):
    return jnp.concatenate(
        [x, jnp.zeros((n - x.shape[0], x.shape[1]), x.dtype)], axis=0)


def _padids(ids, n):
    pad = jnp.full((n - ids.shape[0],), B, jnp.int32)
    return jnp.concatenate([ids, pad]).reshape(n, 1)


def _pad16(x):
    n, d = x.shape
    return jnp.concatenate([x, jnp.zeros((n, 16 - d), jnp.float32)], axis=1)


def _bn_fold(g, b):
    scale = g / jnp.sqrt(1.0 + BN_EPS)
    return scale.reshape(1, -1), b.reshape(1, -1)


def _row(v):
    return v.reshape(1, -1)


def kernel(cell_x, cell_edge_index, cell_batch, drug_x, drug_edge_index,
           drug_batch, params):
    p = params
    ep_cell = _round_up(E_CELL, CHUNK * NW * GROUP)
    ep_drug = _round_up(E_DRUG, CHUNK * NW * GROUP)
    nacc_c = _round_up(N_CELL + PAD_ROWS, NS * 128)
    nacc_d = _round_up(N_DRUG + PAD_ROWS, NS * 128)
    csrc2, cdst2 = _prep_edges(cell_edge_index, N_CELL, ep_cell)
    dsrc2, ddst2 = _prep_edges(drug_edge_index, N_DRUG, ep_drug)
    cx16 = _padrows(_pad16(cell_x), nacc_c)
    dx16 = _padrows(_pad16(drug_x), nacc_d)
    cids = _padids(cell_batch, nacc_c)
    dids = _padids(drug_batch, nacc_d)
    zc1 = jnp.zeros((nacc_c,), jnp.float32)
    zc16 = jnp.zeros((nacc_c, 16), jnp.float32)
    zd16 = jnp.zeros((nacc_d, 16), jnp.float32)

    nch_c = ep_cell // CHUNK
    deg = _sc_degree(nch_c, nacc_c)(cdst2, zc1)
    deg0 = deg[0].reshape(nacc_c, 1)
    deg1 = deg[1].reshape(nacc_c, 1)
    dinv, xs = _tc_dinv_xs(deg0, deg1, cx16)

    agg1 = _sc_rowagg16(nch_c, nacc_c)(csrc2, cdst2, xs, zc16)

    W1p = jnp.zeros((16, 256), jnp.float32).at[:4].set(p['cW1'])
    v = _tc_cell_dense1(agg1[0], agg1[1], xs, dinv, W1p, _row(p['cb1']),
                        p['cW2'])

    vq = v.reshape(nacc_c, 8, 16).transpose(1, 0, 2)
    agg2 = _sc_rowagg16x8(nch_c, nacc_c)(csrc2, cdst2, vq, zc16)
    g_cell = _tc_cell_pool(
        tuple(agg2[e] for e in range(8)), v, dinv, _row(p['cb2']),
        cids)[:B]

    nch_d = ep_drug // CHUNK
    dagg1 = _sc_rowagg16(nch_d, nacc_d)(dsrc2, ddst2, dx16, zd16)
    g1W1p = jnp.zeros((16, 128), jnp.float32).at[:9].set(p['g1W1'])
    s1, o1 = _bn_fold(p['g1Mg'], p['g1Mb'])
    s2, o2 = _bn_fold(p['d1Bg'], p['d1Bb'])
    hd1 = _tc_drug_dense1(
        dagg1[0], dagg1[1], dx16, g1W1p,
        _row(p['g1b1']), s1, o1, p['g1W2'], _row(p['g1b2']), s2, o2)

    hq = hd1.reshape(nacc_d, 8, 16).transpose(1, 0, 2)
    dagg2 = _sc_rowagg16x8(nch_d, nacc_d)(dsrc2, ddst2, hq, zd16)
    s3, o3 = _bn_fold(p['g2Mg'], p['g2Mb'])
    s4, o4 = _bn_fold(p['d2Bg'], p['d2Bb'])
    g_drug = _tc_drug_pool(
        tuple(dagg2[e] for e in range(8)), hd1, p['g2W1'],
        _row(p['g2b1']), s3, o3,
        p['g2W2'], _row(p['g2b2']), s4, o4, dids)[:B]

    cs, co = _bn_fold(p['cBNg'], p['cBNb'])
    ds_, do = _bn_fold(p['dBg'], p['dBb'])
    f1s, f1o = _bn_fold(p['fB1g'], p['fB1b'])
    f2s, f2o = _bn_fold(p['fB2g'], p['fB2b'])
    out = _tc_head(g_cell, g_drug, (
        p['cL1'], _row(p['cL1b']), cs, co, p['cL2'], _row(p['cL2b']),
        p['dL1'], _row(p['dL1b']), ds_, do, p['dL2'], _row(p['dL2b']),
        p['fW1'], _row(p['fb1']), f1s, f1o,
        p['fW2'], _row(p['fb2']), f2s, f2o,
        p['fW3'], _row(p['fb3'])))
    return out.reshape(B)


# speedup vs baseline: 1.4525x; 1.0225x over previous
import functools

import jax
import jax.numpy as jnp
from jax import lax
from jax.experimental import pallas as pl
from jax.experimental.pallas import tpu as pltpu
from jax.experimental.pallas import tpu_sc as plsc

N_CELL = 50000
E_CELL = 800000
N_DRUG = 40000
E_DRUG = 160000
B = 1024
BN_EPS = 1e-5

NC = 2
NS = 16
NW = NC * NS
CHUNK = 128
GROUP = 10
PAD_ROWS = 256


def _mesh():
    return plsc.VectorSubcoreMesh(
        core_axis_name="c", subcore_axis_name="s", num_cores=NC,
        num_subcores=NS)


def _round_up(x, m):
    return (x + m - 1) // m * m


@functools.lru_cache(None)
def _sc_degree(n_chunks, nacc):
    cw = n_chunks // NW
    groups = cw // GROUP

    @functools.partial(
        pl.kernel,
        out_type=jax.ShapeDtypeStruct((NC, nacc), jnp.float32),
        mesh=_mesh(),
        compiler_params=pltpu.CompilerParams(use_tc_tiling_on_sc=False),
        scratch_types=dict(
            didx=pltpu.VMEM((GROUP, CHUNK), jnp.int32),
            ones=pltpu.VMEM((CHUNK,), jnp.float32),
            acc=pltpu.VMEM_SHARED((nacc,), jnp.float32),
            semz=pltpu.SemaphoreType.DMA,
            sems=pltpu.SemaphoreType.DMA,
        ),
    )
    def deg_kernel(dst2d, zeros1, out, didx, ones, acc, semz, sems):
        c = lax.axis_index("c")
        s = lax.axis_index("s")
        w = s * NC + c
        for i in range(CHUNK // 16):
            ones[pl.ds(i * 16, 16)] = jnp.ones((16,), jnp.float32)
        rows_pt = nacc // NS
        base = pl.multiple_of(s * rows_pt, 8)
        pltpu.async_copy(zeros1.at[pl.ds(base, rows_pt)],
                         acc.at[pl.ds(base, rows_pt)], semz).wait()
        plsc.subcore_barrier()

        def body(g, _):
            gbase = w * cw + g * GROUP
            pltpu.sync_copy(dst2d.at[pl.ds(gbase, GROUP)], didx)
            descs = [
                pltpu.async_copy(ones, acc.at[didx.at[j]], sems, add=True)
                for j in range(GROUP)
            ]
            for d in descs:
                d.wait()
            return _

        lax.fori_loop(0, groups, body, None)
        plsc.subcore_barrier()
        pltpu.async_copy(acc.at[pl.ds(base, rows_pt)],
                         out.at[c].at[pl.ds(base, rows_pt)], semz).wait()

    return deg_kernel


@functools.lru_cache(None)
def _sc_rowagg16(n_chunks, nacc):
    cw = n_chunks // NW
    groups = cw // GROUP
    D = 16

    @functools.partial(
        pl.kernel,
        out_type=jax.ShapeDtypeStruct((NC, nacc, D), jnp.float32),
        mesh=_mesh(),
        compiler_params=pltpu.CompilerParams(use_tc_tiling_on_sc=False),
        scratch_types=dict(
            sidx=pltpu.VMEM((GROUP, CHUNK), jnp.int32),
            didx=pltpu.VMEM((GROUP, CHUNK), jnp.int32),
            rows=pltpu.VMEM((GROUP, CHUNK, D), jnp.float32),
            acc=pltpu.VMEM_SHARED((nacc, D), jnp.float32),
            vstage=pltpu.VMEM_SHARED((nacc, D), jnp.float32),
            semz=pltpu.SemaphoreType.DMA,
            semg=pltpu.SemaphoreType.DMA,
            sems=pltpu.SemaphoreType.DMA,
        ),
    )
    def agg16_kernel(src2d, dst2d, vals, zeros2, out,
                     sidx, didx, rows, acc, vstage, semz, semg, sems):
        c = lax.axis_index("c")
        s = lax.axis_index("s")
        w = s * NC + c
        rows_pt = nacc // NS
        base = pl.multiple_of(s * rows_pt, 8)
        pltpu.async_copy(zeros2.at[pl.ds(base, rows_pt)],
                         acc.at[pl.ds(base, rows_pt)], semz).wait()
        pltpu.async_copy(vals.at[pl.ds(base, rows_pt)],
                         vstage.at[pl.ds(base, rows_pt)], semz).wait()
        plsc.subcore_barrier()

        def body(g, _):
            gbase = w * cw + g * GROUP
            pltpu.sync_copy(src2d.at[pl.ds(gbase, GROUP)], sidx)
            pltpu.sync_copy(dst2d.at[pl.ds(gbase, GROUP)], didx)
            gd = [
                pltpu.async_copy(vstage.at[sidx.at[j]], rows.at[j], semg)
                for j in range(GROUP)
            ]
            sd = []
            for j in range(GROUP):
                gd[j].wait()
                sd.append(pltpu.async_copy(rows.at[j], acc.at[didx.at[j]],
                                           sems, add=True))
            for d in sd:
                d.wait()
            return _

        lax.fori_loop(0, groups, body, None)
        plsc.subcore_barrier()
        pltpu.async_copy(acc.at[pl.ds(base, rows_pt)],
                         out.at[c].at[pl.ds(base, rows_pt)], semz).wait()

    return agg16_kernel


@functools.lru_cache(None)
def _sc_rowagg16x8(n_chunks, nacc):
    cw = n_chunks // NS
    groups = cw // GROUP
    D = 16

    @functools.partial(
        pl.kernel,
        out_type=jax.ShapeDtypeStruct((8, nacc, D), jnp.float32),
        mesh=_mesh(),
        compiler_params=pltpu.CompilerParams(use_tc_tiling_on_sc=False),
        scratch_types=dict(
            sidx=pltpu.VMEM((GROUP, CHUNK), jnp.int32),
            didx=pltpu.VMEM((GROUP, CHUNK), jnp.int32),
            rows=pltpu.VMEM((GROUP, CHUNK, D), jnp.float32),
            acc=pltpu.VMEM_SHARED((nacc, D), jnp.float32),
            vstage=pltpu.VMEM_SHARED((nacc, D), jnp.float32),
            semz=pltpu.SemaphoreType.DMA,
            semg=pltpu.SemaphoreType.DMA,
            sems=pltpu.SemaphoreType.DMA,
        ),
    )
    def agg8_kernel(src2d, dst2d, vsplit, zeros2, out,
                    sidx, didx, rows, acc, vstage, semz, semg, sems):
        c = lax.axis_index("c")
        s = lax.axis_index("s")
        rows_pt = nacc // NS
        base = pl.multiple_of(s * rows_pt, 8)

        for step in range(4):
            e = c + 2 * step
            pltpu.async_copy(zeros2.at[pl.ds(base, rows_pt)],
                             acc.at[pl.ds(base, rows_pt)], semz).wait()
            pltpu.async_copy(vsplit.at[e].at[pl.ds(base, rows_pt)],
                             vstage.at[pl.ds(base, rows_pt)], semz).wait()
            plsc.subcore_barrier()

            def body(g, _):
                gbase = s * cw + g * GROUP
                pltpu.sync_copy(src2d.at[pl.ds(gbase, GROUP)], sidx)
                pltpu.sync_copy(dst2d.at[pl.ds(gbase, GROUP)], didx)
                gd = [
                    pltpu.async_copy(vstage.at[sidx.at[j]], rows.at[j],
                                     semg)
                    for j in range(GROUP)
                ]
                sd = []
                for j in range(GROUP):
                    gd[j].wait()
                    sd.append(pltpu.async_copy(rows.at[j],
                                               acc.at[didx.at[j]],
                                               sems, add=True))
                for d in sd:
                    d.wait()
                return _

            lax.fori_loop(0, groups, body, None)
            plsc.subcore_barrier()
            pltpu.async_copy(acc.at[pl.ds(base, rows_pt)],
                             out.at[e].at[pl.ds(base, rows_pt)], semz).wait()
            plsc.subcore_barrier()

    return agg8_kernel


_RB = 1024


def _dot(a, b):
    return jnp.dot(a, b, preferred_element_type=jnp.float32)


def _tc_dinv_xs(p0, p1, xpad):
    n = xpad.shape[0]

    def body(p0_r, p1_r, x_r, dinv_r, xs_r):
        deg = 1.0 + p0_r[...] + p1_r[...]
        dinv = 1.0 / jnp.sqrt(deg)
        dinv_r[...] = dinv
        xs_r[...] = dinv * x_r[...]

    grid = n // _RB
    return pl.pallas_call(
        body,
        grid=(grid,),
        in_specs=[
            pl.BlockSpec((_RB, 1), lambda i: (i, 0)),
            pl.BlockSpec((_RB, 1), lambda i: (i, 0)),
            pl.BlockSpec((_RB, 16), lambda i: (i, 0)),
        ],
        out_specs=[
            pl.BlockSpec((_RB, 1), lambda i: (i, 0)),
            pl.BlockSpec((_RB, 16), lambda i: (i, 0)),
        ],
        out_shape=[
            jax.ShapeDtypeStruct((n, 1), jnp.float32),
            jax.ShapeDtypeStruct((n, 16), jnp.float32),
        ],
    )(p0, p1, xpad)


def _tc_cell_dense1(p0, p1, xs, dinv, W1p, b1, W2):
    n = xs.shape[0]

    def body(p0_r, p1_r, xs_r, dinv_r, w1_r, b1_r, w2_r, v_r):
        dinv = dinv_r[...]
        u = dinv * (p0_r[...] + p1_r[...] + xs_r[...])
        h1 = jax.nn.relu(_dot(u, w1_r[...]) + b1_r[...])
        v_r[...] = dinv * _dot(h1, w2_r[...])

    grid = n // _RB
    return pl.pallas_call(
        body,
        grid=(grid,),
        in_specs=[
            pl.BlockSpec((_RB, 16), lambda i: (i, 0)),
            pl.BlockSpec((_RB, 16), lambda i: (i, 0)),
            pl.BlockSpec((_RB, 16), lambda i: (i, 0)),
            pl.BlockSpec((_RB, 1), lambda i: (i, 0)),
            pl.BlockSpec((16, 256), lambda i: (0, 0)),
            pl.BlockSpec((1, 256), lambda i: (0, 0)),
            pl.BlockSpec((256, 128), lambda i: (0, 0)),
        ],
        out_specs=pl.BlockSpec((_RB, 128), lambda i: (i, 0)),
        out_shape=jax.ShapeDtypeStruct((n, 128), jnp.float32),
    )(p0, p1, xs, dinv, W1p, b1, W2)


def _segmax_update(ids_r, h, out_r):
    ids = ids_r[...]
    lo = ids[0, 0]
    hi = ids[_RB - 1, 0]
    neg = jnp.float32(-jnp.inf)

    def seg_body(j, _):
        seg = lo + j
        m = jnp.where(ids == seg, h, neg)
        red = jnp.max(m, axis=0, keepdims=True)
        cur = out_r[pl.ds(seg, 1), :]
        out_r[pl.ds(seg, 1), :] = jnp.maximum(cur, red)
        return _

    lax.fori_loop(0, hi - lo + 1, seg_body, None)


def _tc_cell_pool(qs, v, dinv, b2, ids):
    q0, q1, q2, q3, q4, q5, q6, q7 = qs
    n = v.shape[0]

    def body(q0_r, q1_r, q2_r, q3_r, q4_r, q5_r, q6_r, q7_r, v_r, dinv_r,
             b2_r, ids_r, out_r):
        i = pl.program_id(0)

        @pl.when(i == 0)
        def _init():
            out_r[...] = jnp.full((B + 8, 128), -jnp.inf, jnp.float32)

        qcat = jnp.concatenate(
            [q0_r[...], q1_r[...], q2_r[...], q3_r[...],
             q4_r[...], q5_r[...], q6_r[...], q7_r[...]], axis=1)
        h2 = dinv_r[...] * (qcat + v_r[...]) + b2_r[...]
        _segmax_update(ids_r, h2, out_r)

    grid = n // _RB
    return pl.pallas_call(
        body,
        grid=(grid,),
        in_specs=[pl.BlockSpec((_RB, 16), lambda i: (i, 0))] * 8 + [
            pl.BlockSpec((_RB, 128), lambda i: (i, 0)),
            pl.BlockSpec((_RB, 1), lambda i: (i, 0)),
            pl.BlockSpec((1, 128), lambda i: (0, 0)),
            pl.BlockSpec((_RB, 1), lambda i: (i, 0)),
        ],
        out_specs=pl.BlockSpec((B + 8, 128), lambda i: (0, 0)),
        out_shape=jax.ShapeDtypeStruct((B + 8, 128), jnp.float32),
    )(q0, q1, q2, q3, q4, q5, q6, q7, v, dinv, b2, ids)


def _tc_drug_dense1(p0, p1, xpad, W1p, b1, s1, o1, W2, b2, s2, o2):
    n = xpad.shape[0]

    def body(p0_r, p1_r, x_r, w1_r, b1_r, s1_r, o1_r, w2_r, b2_r, s2_r,
             o2_r, out_r):
        a = p0_r[...] + p1_r[...] + x_r[...]
        t = _dot(a, w1_r[...]) + b1_r[...]
        t = t * s1_r[...] + o1_r[...]
        t = jax.nn.relu(t)
        t = _dot(t, w2_r[...]) + b2_r[...]
        t = jax.nn.relu(t)
        out_r[...] = t * s2_r[...] + o2_r[...]

    grid = n // _RB
    return pl.pallas_call(
        body,
        grid=(grid,),
        in_specs=[
            pl.BlockSpec((_RB, 16), lambda i: (i, 0)),
            pl.BlockSpec((_RB, 16), lambda i: (i, 0)),
            pl.BlockSpec((_RB, 16), lambda i: (i, 0)),
            pl.BlockSpec((16, 128), lambda i: (0, 0)),
            pl.BlockSpec((1, 128), lambda i: (0, 0)),
            pl.BlockSpec((1, 128), lambda i: (0, 0)),
            pl.BlockSpec((1, 128), lambda i: (0, 0)),
            pl.BlockSpec((128, 128), lambda i: (0, 0)),
            pl.BlockSpec((1, 128), lambda i: (0, 0)),
            pl.BlockSpec((1, 128), lambda i: (0, 0)),
            pl.BlockSpec((1, 128), lambda i: (0, 0)),
        ],
        out_specs=pl.BlockSpec((_RB, 128), lambda i: (i, 0)),
        out_shape=jax.ShapeDtypeStruct((n, 128), jnp.float32),
    )(p0, p1, xpad, W1p, b1, s1, o1, W2, b2, s2, o2)


def _tc_drug_pool(qs, hd1, W1, b1, s1, o1, W2, b2, s2, o2, ids):
    q0, q1, q2, q3, q4, q5, q6, q7 = qs
    n = hd1.shape[0]

    def body(q0_r, q1_r, q2_r, q3_r, q4_r, q5_r, q6_r, q7_r, h_r, w1_r,
             b1_r, s1_r, o1_r, w2_r, b2_r, s2_r, o2_r, ids_r, out_r):
        i = pl.program_id(0)

        @pl.when(i == 0)
        def _init():
            out_r[...] = jnp.full((B + 8, 128), -jnp.inf, jnp.float32)

        a = jnp.concatenate(
            [q0_r[...], q1_r[...], q2_r[...], q3_r[...],
             q4_r[...], q5_r[...], q6_r[...], q7_r[...]],
            axis=1) + h_r[...]
        t = _dot(a, w1_r[...]) + b1_r[...]
        t = t * s1_r[...] + o1_r[...]
        t = jax.nn.relu(t)
        t = _dot(t, w2_r[...]) + b2_r[...]
        t = jax.nn.relu(t)
        h2 = t * s2_r[...] + o2_r[...]
        _segmax_update(ids_r, h2, out_r)

    grid = n // _RB
    return pl.pallas_call(
        body,
        grid=(grid,),
        in_specs=[pl.BlockSpec((_RB, 16), lambda i: (i, 0))] * 8 + [
            pl.BlockSpec((_RB, 128), lambda i: (i, 0)),
            pl.BlockSpec((128, 128), lambda i: (0, 0)),
            pl.BlockSpec((1, 128), lambda i: (0, 0)),
            pl.BlockSpec((1, 128), lambda i: (0, 0)),
            pl.BlockSpec((1, 128), lambda i: (0, 0)),
            pl.BlockSpec((128, 128), lambda i: (0, 0)),
            pl.BlockSpec((1, 128), lambda i: (0, 0)),
            pl.BlockSpec((1, 128), lambda i: (0, 0)),
            pl.BlockSpec((1, 128), lambda i: (0, 0)),
            pl.BlockSpec((_RB, 1), lambda i: (i, 0)),
        ],
        out_specs=pl.BlockSpec((B + 8, 128), lambda i: (0, 0)),
        out_shape=jax.ShapeDtypeStruct((B + 8, 128), jnp.float32),
    )(q0, q1, q2, q3, q4, q5, q6, q7, hd1, W1, b1, s1, o1, W2, b2, s2, o2,
      ids)


def _tc_head(gc, gd, wts):
    (cL1, cL1b, cBNs, cBNo, cL2, cL2b,
     dL1, dL1b, dBs, dBo, dL2, dL2b,
     fW1, fb1, fB1s, fB1o, fW2, fb2, fB2s, fB2o, fW3, fb3) = wts

    def body(gc_r, gd_r, cL1_r, cL1b_r, cBNs_r, cBNo_r, cL2_r, cL2b_r,
             dL1_r, dL1b_r, dBs_r, dBo_r, dL2_r, dL2b_r,
             fW1_r, fb1_r, fB1s_r, fB1o_r, fW2_r, fb2_r, fB2s_r, fB2o_r,
             fW3_r, fb3_r, out_r):
        g = _dot(gc_r[...], cL1_r[...]) + cL1b_r[...]
        g = g * cBNs_r[...] + cBNo_r[...]
        g = jax.nn.relu(g)
        g = _dot(g, cL2_r[...]) + cL2b_r[...]
        cell_emb = jax.nn.relu(g)

        g = _dot(gd_r[...], dL1_r[...]) + dL1b_r[...]
        g = g * dBs_r[...] + dBo_r[...]
        g = jax.nn.relu(g)
        g = _dot(g, dL2_r[...]) + dL2b_r[...]
        drug_emb = jax.nn.relu(g)

        z = jnp.concatenate([cell_emb, drug_emb], axis=1)
        z = _dot(z, fW1_r[...]) + fb1_r[...]
        z = z * fB1s_r[...] + fB1o_r[...]
        z = jnp.where(z > 0, z, jnp.exp(z) - 1.0)
        z = _dot(z, fW2_r[...]) + fb2_r[...]
        z = z * fB2s_r[...] + fB2o_r[...]
        z = jnp.where(z > 0, z, jnp.exp(z) - 1.0)
        out_r[...] = _dot(z, fW3_r[...]) + fb3_r[...]

    def _spec(a):
        zero = (0,) * a.ndim
        return pl.BlockSpec(a.shape, lambda i, z=zero: z)

    specs = [_spec(a)
             for a in (gc, gd, cL1, cL1b, cBNs, cBNo, cL2, cL2b,
                       dL1, dL1b, dBs, dBo, dL2, dL2b,
                       fW1, fb1, fB1s, fB1o, fW2, fb2, fB2s, fB2o, fW3, fb3)]
    return pl.pallas_call(
        body,
        grid=(1,),
        in_specs=specs,
        out_specs=pl.BlockSpec((B, 1), lambda i: (0, 0)),
        out_shape=jax.ShapeDtypeStruct((B, 1), jnp.float32),
    )(gc, gd, cL1, cL1b, cBNs, cBNo, cL2, cL2b,
      dL1, dL1b, dBs, dBo, dL2, dL2b,
      fW1, fb1, fB1s, fB1o, fW2, fb2, fB2s, fB2o, fW3, fb3)


def _prep_edges(ei, n_nodes, e_pad):
    src = ei[0]
    dst = ei[1]
    e = src.shape[0]
    pad_n = e_pad - e
    src_p = jnp.concatenate([src, jnp.zeros((pad_n,), jnp.int32)])
    dst_p = jnp.concatenate(
        [dst, n_nodes + (jnp.arange(pad_n, dtype=jnp.int32) % PAD_ROWS)])
    return src_p.reshape(e_pad // CHUNK, CHUNK), dst_p.reshape(
        e_pad // CHUNK, CHUNK)


def _padrows(x, n):
    return jnp.concatenate(
        [x, jnp.zeros((n - x.shape[0], x.shape[1]), x.dtype)], axis=0)


def _padids(ids, n):
    pad = jnp.full((n - ids.shape[0],), B, jnp.int32)
    return jnp.concatenate([ids, pad]).reshape(n, 1)


def _pad16(x):
    n, d = x.shape
    return jnp.concatenate([x, jnp.zeros((n, 16 - d), jnp.float32)], axis=1)


def _bn_fold(g, b):
    scale = g / jnp.sqrt(1.0 + BN_EPS)
    return scale.reshape(1, -1), b.reshape(1, -1)


def _row(v):
    return v.reshape(1, -1)


def kernel(cell_x, cell_edge_index, cell_batch, drug_x, drug_edge_index,
           drug_batch, params):
    p = params
    ep_cell = _round_up(E_CELL, CHUNK * NW * GROUP)
    ep_drug = _round_up(E_DRUG, CHUNK * NW * GROUP)
    nacc_c = _round_up(N_CELL + PAD_ROWS, NS * 128)
    nacc_d = _round_up(N_DRUG + PAD_ROWS, NS * 128)
    csrc2, cdst2 = _prep_edges(cell_edge_index, N_CELL, ep_cell)
    dsrc2, ddst2 = _prep_edges(drug_edge_index, N_DRUG, ep_drug)
    cx16 = _padrows(_pad16(cell_x), nacc_c)
    dx16 = _padrows(_pad16(drug_x), nacc_d)
    cids = _padids(cell_batch, nacc_c)
    dids = _padids(drug_batch, nacc_d)
    zc1 = jnp.zeros((nacc_c,), jnp.float32)
    zc16 = jnp.zeros((nacc_c, 16), jnp.float32)
    zd16 = jnp.zeros((nacc_d, 16), jnp.float32)

    nch_c = ep_cell // CHUNK
    deg = _sc_degree(nch_c, nacc_c)(cdst2, zc1)
    deg0 = deg[0].reshape(nacc_c, 1)
    deg1 = deg[1].reshape(nacc_c, 1)
    dinv, xs = _tc_dinv_xs(deg0, deg1, cx16)

    agg1 = _sc_rowagg16(nch_c, nacc_c)(csrc2, cdst2, xs, zc16)

    W1p = jnp.zeros((16, 256), jnp.float32).at[:4].set(p['cW1'])
    v = _tc_cell_dense1(agg1[0], agg1[1], xs, dinv, W1p, _row(p['cb1']),
                        p['cW2'])

    vq = v.reshape(nacc_c, 8, 16).transpose(1, 0, 2)
    agg2 = _sc_rowagg16x8(nch_c, nacc_c)(csrc2, cdst2, vq, zc16)
    g_cell = _tc_cell_pool(
        tuple(agg2[e] for e in range(8)), v, dinv, _row(p['cb2']),
        cids)[:B]

    nch_d = ep_drug // CHUNK
    dagg1 = _sc_rowagg16(nch_d, nacc_d)(dsrc2, ddst2, dx16, zd16)
    g1W1p = jnp.zeros((16, 128), jnp.float32).at[:9].set(p['g1W1'])
    s1, o1 = _bn_fold(p['g1Mg'], p['g1Mb'])
    s2, o2 = _bn_fold(p['d1Bg'], p['d1Bb'])
    hd1 = _tc_drug_dense1(
        dagg1[0], dagg1[1], dx16, g1W1p,
        _row(p['g1b1']), s1, o1, p['g1W2'], _row(p['g1b2']), s2, o2)

    hq = hd1.reshape(nacc_d, 8, 16).transpose(1, 0, 2)
    dagg2 = _sc_rowagg16x8(nch_d, nacc_d)(dsrc2, ddst2, hq, zd16)
    s3, o3 = _bn_fold(p['g2Mg'], p['g2Mb'])
    s4, o4 = _bn_fold(p['d2Bg'], p['d2Bb'])
    g_drug = _tc_drug_pool(
        tuple(dagg2[e] for e in range(8)), hd1, p['g2W1'],
        _row(p['g2b1']), s3, o3,
        p['g2W2'], _row(p['g2b2']), s4, o4, dids)[:B]

    cs, co = _bn_fold(p['cBNg'], p['cBNb'])
    ds_, do = _bn_fold(p['dBg'], p['dBb'])
    f1s, f1o = _bn_fold(p['fB1g'], p['fB1b'])
    f2s, f2o = _bn_fold(p['fB2g'], p['fB2b'])
    out = _tc_head(g_cell, g_drug, (
        p['cL1'], _row(p['cL1b']), cs, co, p['cL2'], _row(p['cL2b']),
        p['dL1'], _row(p['dL1b']), ds_, do, p['dL2'], _row(p['dL2b']),
        p['fW1'], _row(p['fb1']), f1s, f1o,
        p['fW2'], _row(p['fb2']), f2s, f2o,
        p['fW3'], _row(p['fb3'])))
    return out.reshape(B)


# speedup vs baseline: 1.4791x; 1.0183x over previous
import functools

import jax
import jax.numpy as jnp
from jax import lax
from jax.experimental import pallas as pl
from jax.experimental.pallas import tpu as pltpu
from jax.experimental.pallas import tpu_sc as plsc

N_CELL = 50000
E_CELL = 800000
N_DRUG = 40000
E_DRUG = 160000
B = 1024
BN_EPS = 1e-5

NC = 2
NS = 16
NW = NC * NS
CHUNK = 128
GROUP = 10
PAD_ROWS = 256


def _mesh():
    return plsc.VectorSubcoreMesh(
        core_axis_name="c", subcore_axis_name="s", num_cores=NC,
        num_subcores=NS)


def _round_up(x, m):
    return (x + m - 1) // m * m


@functools.lru_cache(None)
def _sc_degree(n_chunks, nacc):
    cw = n_chunks // NW
    supers = cw // (2 * GROUP)

    @functools.partial(
        pl.kernel,
        out_type=jax.ShapeDtypeStruct((NC, nacc), jnp.float32),
        mesh=_mesh(),
        compiler_params=pltpu.CompilerParams(use_tc_tiling_on_sc=False),
        scratch_types=dict(
            didx=pltpu.VMEM((2 * GROUP, CHUNK), jnp.int32),
            ones=pltpu.VMEM((CHUNK,), jnp.float32),
            acc=pltpu.VMEM_SHARED((nacc,), jnp.float32),
            semz=pltpu.SemaphoreType.DMA,
            sems=pltpu.SemaphoreType.DMA,
        ),
    )
    def deg_kernel(dst2d, zeros1, out, didx, ones, acc, semz, sems):
        c = lax.axis_index("c")
        s = lax.axis_index("s")
        w = s * NC + c
        for i in range(CHUNK // 16):
            ones[pl.ds(i * 16, 16)] = jnp.ones((16,), jnp.float32)
        rows_pt = nacc // NS
        base = pl.multiple_of(s * rows_pt, 8)
        pltpu.async_copy(zeros1.at[pl.ds(base, rows_pt)],
                         acc.at[pl.ds(base, rows_pt)], semz).wait()
        plsc.subcore_barrier()

        def body(g, _):
            gbase = w * cw + g * 2 * GROUP
            pltpu.sync_copy(dst2d.at[pl.ds(gbase, 2 * GROUP)], didx)
            descs = [
                pltpu.async_copy(ones, acc.at[didx.at[j]], sems, add=True)
                for j in range(2 * GROUP)
            ]
            for d in descs:
                d.wait()
            return _

        lax.fori_loop(0, supers, body, None)
        plsc.subcore_barrier()
        pltpu.async_copy(acc.at[pl.ds(base, rows_pt)],
                         out.at[c].at[pl.ds(base, rows_pt)], semz).wait()

    return deg_kernel


@functools.lru_cache(None)
def _sc_rowagg16(n_chunks, nacc):
    cw = n_chunks // NW
    supers = cw // (2 * GROUP)
    D = 16

    @functools.partial(
        pl.kernel,
        out_type=jax.ShapeDtypeStruct((NC, nacc, D), jnp.float32),
        mesh=_mesh(),
        compiler_params=pltpu.CompilerParams(use_tc_tiling_on_sc=False),
        scratch_types=dict(
            sidx=pltpu.VMEM((2 * GROUP, CHUNK), jnp.int32),
            didx=pltpu.VMEM((2 * GROUP, CHUNK), jnp.int32),
            rows=pltpu.VMEM((GROUP, CHUNK, D), jnp.float32),
            acc=pltpu.VMEM_SHARED((nacc, D), jnp.float32),
            vstage=pltpu.VMEM_SHARED((nacc, D), jnp.float32),
            semz=pltpu.SemaphoreType.DMA,
            semg=pltpu.SemaphoreType.DMA,
            sems=pltpu.SemaphoreType.DMA,
        ),
    )
    def agg16_kernel(src2d, dst2d, vals, zeros2, out,
                     sidx, didx, rows, acc, vstage, semz, semg, sems):
        c = lax.axis_index("c")
        s = lax.axis_index("s")
        w = s * NC + c
        rows_pt = nacc // NS
        base = pl.multiple_of(s * rows_pt, 8)
        pltpu.async_copy(zeros2.at[pl.ds(base, rows_pt)],
                         acc.at[pl.ds(base, rows_pt)], semz).wait()
        pltpu.async_copy(vals.at[pl.ds(base, rows_pt)],
                         vstage.at[pl.ds(base, rows_pt)], semz).wait()
        plsc.subcore_barrier()

        def body(g, _):
            gbase = w * cw + g * 2 * GROUP
            pltpu.sync_copy(src2d.at[pl.ds(gbase, 2 * GROUP)], sidx)
            pltpu.sync_copy(dst2d.at[pl.ds(gbase, 2 * GROUP)], didx)
            for h in range(2):
                gd = [
                    pltpu.async_copy(vstage.at[sidx.at[h * GROUP + j]],
                                     rows.at[j], semg)
                    for j in range(GROUP)
                ]
                sd = []
                for j in range(GROUP):
                    gd[j].wait()
                    sd.append(pltpu.async_copy(
                        rows.at[j], acc.at[didx.at[h * GROUP + j]],
                        sems, add=True))
                for d in sd:
                    d.wait()
            return _

        lax.fori_loop(0, supers, body, None)
        plsc.subcore_barrier()
        pltpu.async_copy(acc.at[pl.ds(base, rows_pt)],
                         out.at[c].at[pl.ds(base, rows_pt)], semz).wait()

    return agg16_kernel


@functools.lru_cache(None)
def _sc_rowagg16x8(n_chunks, nacc):
    cw = n_chunks // NS
    supers = cw // (2 * GROUP)
    D = 16

    @functools.partial(
        pl.kernel,
        out_type=jax.ShapeDtypeStruct((8, nacc, D), jnp.float32),
        mesh=_mesh(),
        compiler_params=pltpu.CompilerParams(use_tc_tiling_on_sc=False),
        scratch_types=dict(
            sidx=pltpu.VMEM((2 * GROUP, CHUNK), jnp.int32),
            didx=pltpu.VMEM((2 * GROUP, CHUNK), jnp.int32),
            rows=pltpu.VMEM((GROUP, CHUNK, D), jnp.float32),
            acc=pltpu.VMEM_SHARED((nacc, D), jnp.float32),
            vstage=pltpu.VMEM_SHARED((nacc, D), jnp.float32),
            semz=pltpu.SemaphoreType.DMA,
            semg=pltpu.SemaphoreType.DMA,
            sems=pltpu.SemaphoreType.DMA,
        ),
    )
    def agg8_kernel(src2d, dst2d, vsplit, zeros2, out,
                    sidx, didx, rows, acc, vstage, semz, semg, sems):
        c = lax.axis_index("c")
        s = lax.axis_index("s")
        rows_pt = nacc // NS
        base = pl.multiple_of(s * rows_pt, 8)

        for step in range(4):
            e = c + 2 * step
            pltpu.async_copy(zeros2.at[pl.ds(base, rows_pt)],
                             acc.at[pl.ds(base, rows_pt)], semz).wait()
            pltpu.async_copy(vsplit.at[e].at[pl.ds(base, rows_pt)],
                             vstage.at[pl.ds(base, rows_pt)], semz).wait()
            plsc.subcore_barrier()

            def body(g, _):
                gbase = s * cw + g * 2 * GROUP
                pltpu.sync_copy(src2d.at[pl.ds(gbase, 2 * GROUP)], sidx)
                pltpu.sync_copy(dst2d.at[pl.ds(gbase, 2 * GROUP)], didx)
                for h in range(2):
                    gd = [
                        pltpu.async_copy(
                            vstage.at[sidx.at[h * GROUP + j]],
                            rows.at[j], semg)
                        for j in range(GROUP)
                    ]
                    sd = []
                    for j in range(GROUP):
                        gd[j].wait()
                        sd.append(pltpu.async_copy(
                            rows.at[j], acc.at[didx.at[h * GROUP + j]],
                            sems, add=True))
                    for d in sd:
                        d.wait()
                return _

            lax.fori_loop(0, supers, body, None)
            plsc.subcore_barrier()
            pltpu.async_copy(acc.at[pl.ds(base, rows_pt)],
                             out.at[e].at[pl.ds(base, rows_pt)], semz).wait()
            plsc.subcore_barrier()

    return agg8_kernel


_RB = 1024


def _dot(a, b):
    return jnp.dot(a, b, preferred_element_type=jnp.float32)


def _tc_dinv_xs(p0, p1, xpad):
    n = xpad.shape[0]

    def body(p0_r, p1_r, x_r, dinv_r, xs_r):
        deg = 1.0 + p0_r[...] + p1_r[...]
        dinv = 1.0 / jnp.sqrt(deg)
        dinv_r[...] = dinv
        xs_r[...] = dinv * x_r[...]

    grid = n // _RB
    return pl.pallas_call(
        body,
        grid=(grid,),
        in_specs=[
            pl.BlockSpec((_RB, 1), lambda i: (i, 0)),
            pl.BlockSpec((_RB, 1), lambda i: (i, 0)),
            pl.BlockSpec((_RB, 16), lambda i: (i, 0)),
        ],
        out_specs=[
            pl.BlockSpec((_RB, 1), lambda i: (i, 0)),
            pl.BlockSpec((_RB, 16), lambda i: (i, 0)),
        ],
        out_shape=[
            jax.ShapeDtypeStruct((n, 1), jnp.float32),
            jax.ShapeDtypeStruct((n, 16), jnp.float32),
        ],
    )(p0, p1, xpad)


def _tc_cell_dense1(p0, p1, xs, dinv, W1p, b1, W2):
    n = xs.shape[0]

    def body(p0_r, p1_r, xs_r, dinv_r, w1_r, b1_r, w2_r, v_r):
        dinv = dinv_r[...]
        u = dinv * (p0_r[...] + p1_r[...] + xs_r[...])
        h1 = jax.nn.relu(_dot(u, w1_r[...]) + b1_r[...])
        v_r[...] = dinv * _dot(h1, w2_r[...])

    grid = n // _RB
    return pl.pallas_call(
        body,
        grid=(grid,),
        in_specs=[
            pl.BlockSpec((_RB, 16), lambda i: (i, 0)),
            pl.BlockSpec((_RB, 16), lambda i: (i, 0)),
            pl.BlockSpec((_RB, 16), lambda i: (i, 0)),
            pl.BlockSpec((_RB, 1), lambda i: (i, 0)),
            pl.BlockSpec((16, 256), lambda i: (0, 0)),
            pl.BlockSpec((1, 256), lambda i: (0, 0)),
            pl.BlockSpec((256, 128), lambda i: (0, 0)),
        ],
        out_specs=pl.BlockSpec((_RB, 128), lambda i: (i, 0)),
        out_shape=jax.ShapeDtypeStruct((n, 128), jnp.float32),
    )(p0, p1, xs, dinv, W1p, b1, W2)


def _segmax_update(ids_r, h, out_r):
    ids = ids_r[...]
    lo = ids[0, 0]
    hi = ids[_RB - 1, 0]
    neg = jnp.float32(-jnp.inf)

    def seg_body(j, _):
        seg = lo + j
        m = jnp.where(ids == seg, h, neg)
        red = jnp.max(m, axis=0, keepdims=True)
        cur = out_r[pl.ds(seg, 1), :]
        out_r[pl.ds(seg, 1), :] = jnp.maximum(cur, red)
        return _

    lax.fori_loop(0, hi - lo + 1, seg_body, None)


def _tc_cell_pool(qs, v, dinv, b2, ids):
    q0, q1, q2, q3, q4, q5, q6, q7 = qs
    n = v.shape[0]

    def body(q0_r, q1_r, q2_r, q3_r, q4_r, q5_r, q6_r, q7_r, v_r, dinv_r,
             b2_r, ids_r, out_r):
        i = pl.program_id(0)

        @pl.when(i == 0)
        def _init():
            out_r[...] = jnp.full((B + 8, 128), -jnp.inf, jnp.float32)

        qcat = jnp.concatenate(
            [q0_r[...], q1_r[...], q2_r[...], q3_r[...],
             q4_r[...], q5_r[...], q6_r[...], q7_r[...]], axis=1)
        h2 = dinv_r[...] * (qcat + v_r[...]) + b2_r[...]
        _segmax_update(ids_r, h2, out_r)

    grid = n // _RB
    return pl.pallas_call(
        body,
        grid=(grid,),
        in_specs=[pl.BlockSpec((_RB, 16), lambda i: (i, 0))] * 8 + [
            pl.BlockSpec((_RB, 128), lambda i: (i, 0)),
            pl.BlockSpec((_RB, 1), lambda i: (i, 0)),
            pl.BlockSpec((1, 128), lambda i: (0, 0)),
            pl.BlockSpec((_RB, 1), lambda i: (i, 0)),
        ],
        out_specs=pl.BlockSpec((B + 8, 128), lambda i: (0, 0)),
        out_shape=jax.ShapeDtypeStruct((B + 8, 128), jnp.float32),
    )(q0, q1, q2, q3, q4, q5, q6, q7, v, dinv, b2, ids)


def _tc_drug_dense1(p0, p1, xpad, W1p, b1, s1, o1, W2, b2, s2, o2):
    n = xpad.shape[0]

    def body(p0_r, p1_r, x_r, w1_r, b1_r, s1_r, o1_r, w2_r, b2_r, s2_r,
             o2_r, out_r):
        a = p0_r[...] + p1_r[...] + x_r[...]
        t = _dot(a, w1_r[...]) + b1_r[...]
        t = t * s1_r[...] + o1_r[...]
        t = jax.nn.relu(t)
        t = _dot(t, w2_r[...]) + b2_r[...]
        t = jax.nn.relu(t)
        out_r[...] = t * s2_r[...] + o2_r[...]

    grid = n // _RB
    return pl.pallas_call(
        body,
        grid=(grid,),
        in_specs=[
            pl.BlockSpec((_RB, 16), lambda i: (i, 0)),
            pl.BlockSpec((_RB, 16), lambda i: (i, 0)),
            pl.BlockSpec((_RB, 16), lambda i: (i, 0)),
            pl.BlockSpec((16, 128), lambda i: (0, 0)),
            pl.BlockSpec((1, 128), lambda i: (0, 0)),
            pl.BlockSpec((1, 128), lambda i: (0, 0)),
            pl.BlockSpec((1, 128), lambda i: (0, 0)),
            pl.BlockSpec((128, 128), lambda i: (0, 0)),
            pl.BlockSpec((1, 128), lambda i: (0, 0)),
            pl.BlockSpec((1, 128), lambda i: (0, 0)),
            pl.BlockSpec((1, 128), lambda i: (0, 0)),
        ],
        out_specs=pl.BlockSpec((_RB, 128), lambda i: (i, 0)),
        out_shape=jax.ShapeDtypeStruct((n, 128), jnp.float32),
    )(p0, p1, xpad, W1p, b1, s1, o1, W2, b2, s2, o2)


def _tc_drug_pool(qs, hd1, W1, b1, s1, o1, W2, b2, s2, o2, ids):
    q0, q1, q2, q3, q4, q5, q6, q7 = qs
    n = hd1.shape[0]

    def body(q0_r, q1_r, q2_r, q3_r, q4_r, q5_r, q6_r, q7_r, h_r, w1_r,
             b1_r, s1_r, o1_r, w2_r, b2_r, s2_r, o2_r, ids_r, out_r):
        i = pl.program_id(0)

        @pl.when(i == 0)
        def _init():
            out_r[...] = jnp.full((B + 8, 128), -jnp.inf, jnp.float32)

        a = jnp.concatenate(
            [q0_r[...], q1_r[...], q2_r[...], q3_r[...],
             q4_r[...], q5_r[...], q6_r[...], q7_r[...]],
            axis=1) + h_r[...]
        t = _dot(a, w1_r[...]) + b1_r[...]
        t = t * s1_r[...] + o1_r[...]
        t = jax.nn.relu(t)
        t = _dot(t, w2_r[...]) + b2_r[...]
        t = jax.nn.relu(t)
        h2 = t * s2_r[...] + o2_r[...]
        _segmax_update(ids_r, h2, out_r)

    grid = n // _RB
    return pl.pallas_call(
        body,
        grid=(grid,),
        in_specs=[pl.BlockSpec((_RB, 16), lambda i: (i, 0))] * 8 + [
            pl.BlockSpec((_RB, 128), lambda i: (i, 0)),
            pl.BlockSpec((128, 128), lambda i: (0, 0)),
            pl.BlockSpec((1, 128), lambda i: (0, 0)),
            pl.BlockSpec((1, 128), lambda i: (0, 0)),
            pl.BlockSpec((1, 128), lambda i: (0, 0)),
            pl.BlockSpec((128, 128), lambda i: (0, 0)),
            pl.BlockSpec((1, 128), lambda i: (0, 0)),
            pl.BlockSpec((1, 128), lambda i: (0, 0)),
            pl.BlockSpec((1, 128), lambda i: (0, 0)),
            pl.BlockSpec((_RB, 1), lambda i: (i, 0)),
        ],
        out_specs=pl.BlockSpec((B + 8, 128), lambda i: (0, 0)),
        out_shape=jax.ShapeDtypeStruct((B + 8, 128), jnp.float32),
    )(q0, q1, q2, q3, q4, q5, q6, q7, hd1, W1, b1, s1, o1, W2, b2, s2, o2,
      ids)


def _tc_head(gc, gd, wts):
    (cL1, cL1b, cBNs, cBNo, cL2, cL2b,
     dL1, dL1b, dBs, dBo, dL2, dL2b,
     fW1, fb1, fB1s, fB1o, fW2, fb2, fB2s, fB2o, fW3, fb3) = wts

    def body(gc_r, gd_r, cL1_r, cL1b_r, cBNs_r, cBNo_r, cL2_r, cL2b_r,
             dL1_r, dL1b_r, dBs_r, dBo_r, dL2_r, dL2b_r,
             fW1_r, fb1_r, fB1s_r, fB1o_r, fW2_r, fb2_r, fB2s_r, fB2o_r,
             fW3_r, fb3_r, out_r):
        g = _dot(gc_r[...], cL1_r[...]) + cL1b_r[...]
        g = g * cBNs_r[...] + cBNo_r[...]
        g = jax.nn.relu(g)
        g = _dot(g, cL2_r[...]) + cL2b_r[...]
        cell_emb = jax.nn.relu(g)

        g = _dot(gd_r[...], dL1_r[...]) + dL1b_r[...]
        g = g * dBs_r[...] + dBo_r[...]
        g = jax.nn.relu(g)
        g = _dot(g, dL2_r[...]) + dL2b_r[...]
        drug_emb = jax.nn.relu(g)

        z = jnp.concatenate([cell_emb, drug_emb], axis=1)
        z = _dot(z, fW1_r[...]) + fb1_r[...]
        z = z * fB1s_r[...] + fB1o_r[...]
        z = jnp.where(z > 0, z, jnp.exp(z) - 1.0)
        z = _dot(z, fW2_r[...]) + fb2_r[...]
        z = z * fB2s_r[...] + fB2o_r[...]
        z = jnp.where(z > 0, z, jnp.exp(z) - 1.0)
        out_r[...] = _dot(z, fW3_r[...]) + fb3_r[...]

    def _spec(a):
        zero = (0,) * a.ndim
        return pl.BlockSpec(a.shape, lambda i, z=zero: z)

    specs = [_spec(a)
             for a in (gc, gd, cL1, cL1b, cBNs, cBNo, cL2, cL2b,
                       dL1, dL1b, dBs, dBo, dL2, dL2b,
                       fW1, fb1, fB1s, fB1o, fW2, fb2, fB2s, fB2o, fW3, fb3)]
    return pl.pallas_call(
        body,
        grid=(1,),
        in_specs=specs,
        out_specs=pl.BlockSpec((B, 1), lambda i: (0, 0)),
        out_shape=jax.ShapeDtypeStruct((B, 1), jnp.float32),
    )(gc, gd, cL1, cL1b, cBNs, cBNo, cL2, cL2b,
      dL1, dL1b, dBs, dBo, dL2, dL2b,
      fW1, fb1, fB1s, fB1o, fW2, fb2, fB2s, fB2o, fW3, fb3)


def _prep_edges(ei, n_nodes, e_pad):
    src = ei[0]
    dst = ei[1]
    e = src.shape[0]
    pad_n = e_pad - e
    src_p = jnp.concatenate([src, jnp.zeros((pad_n,), jnp.int32)])
    dst_p = jnp.concatenate(
        [dst, n_nodes + (jnp.arange(pad_n, dtype=jnp.int32) % PAD_ROWS)])
    return src_p.reshape(e_pad // CHUNK, CHUNK), dst_p.reshape(
        e_pad // CHUNK, CHUNK)


def _padrows(x, n):
    return jnp.concatenate(
        [x, jnp.zeros((n - x.shape[0], x.shape[1]), x.dtype)], axis=0)


def _padids(ids, n):
    pad = jnp.full((n - ids.shape[0],), B, jnp.int32)
    return jnp.concatenate([ids, pad]).reshape(n, 1)


def _pad16(x):
    n, d = x.shape
    return jnp.concatenate([x, jnp.zeros((n, 16 - d), jnp.float32)], axis=1)


def _bn_fold(g, b):
    scale = g / jnp.sqrt(1.0 + BN_EPS)
    return scale.reshape(1, -1), b.reshape(1, -1)


def _row(v):
    return v.reshape(1, -1)


def kernel(cell_x, cell_edge_index, cell_batch, drug_x, drug_edge_index,
           drug_batch, params):
    p = params
    ep_cell = _round_up(E_CELL, CHUNK * NW * GROUP)
    ep_drug = _round_up(E_DRUG, CHUNK * NW * GROUP)
    nacc_c = _round_up(N_CELL + PAD_ROWS, NS * 128)
    nacc_d = _round_up(N_DRUG + PAD_ROWS, NS * 128)
    csrc2, cdst2 = _prep_edges(cell_edge_index, N_CELL, ep_cell)
    dsrc2, ddst2 = _prep_edges(drug_edge_index, N_DRUG, ep_drug)
    cx16 = _padrows(_pad16(cell_x), nacc_c)
    dx16 = _padrows(_pad16(drug_x), nacc_d)
    cids = _padids(cell_batch, nacc_c)
    dids = _padids(drug_batch, nacc_d)
    zc1 = jnp.zeros((nacc_c,), jnp.float32)
    zc16 = jnp.zeros((nacc_c, 16), jnp.float32)
    zd16 = jnp.zeros((nacc_d, 16), jnp.float32)

    nch_c = ep_cell // CHUNK
    deg = _sc_degree(nch_c, nacc_c)(cdst2, zc1)
    deg0 = deg[0].reshape(nacc_c, 1)
    deg1 = deg[1].reshape(nacc_c, 1)
    dinv, xs = _tc_dinv_xs(deg0, deg1, cx16)

    agg1 = _sc_rowagg16(nch_c, nacc_c)(csrc2, cdst2, xs, zc16)

    W1p = jnp.zeros((16, 256), jnp.float32).at[:4].set(p['cW1'])
    v = _tc_cell_dense1(agg1[0], agg1[1], xs, dinv, W1p, _row(p['cb1']),
                        p['cW2'])

    vq = v.reshape(nacc_c, 8, 16).transpose(1, 0, 2)
    agg2 = _sc_rowagg16x8(nch_c, nacc_c)(csrc2, cdst2, vq, zc16)
    g_cell = _tc_cell_pool(
        tuple(agg2[e] for e in range(8)), v, dinv, _row(p['cb2']),
        cids)[:B]

    nch_d = ep_drug // CHUNK
    dagg1 = _sc_rowagg16(nch_d, nacc_d)(dsrc2, ddst2, dx16, zd16)
    g1W1p = jnp.zeros((16, 128), jnp.float32).at[:9].set(p['g1W1'])
    s1, o1 = _bn_fold(p['g1Mg'], p['g1Mb'])
    s2, o2 = _bn_fold(p['d1Bg'], p['d1Bb'])
    hd1 = _tc_drug_dense1(
        dagg1[0], dagg1[1], dx16, g1W1p,
        _row(p['g1b1']), s1, o1, p['g1W2'], _row(p['g1b2']), s2, o2)

    hq = hd1.reshape(nacc_d, 8, 16).transpose(1, 0, 2)
    dagg2 = _sc_rowagg16x8(nch_d, nacc_d)(dsrc2, ddst2, hq, zd16)
    s3, o3 = _bn_fold(p['g2Mg'], p['g2Mb'])
    s4, o4 = _bn_fold(p['d2Bg'], p['d2Bb'])
    g_drug = _tc_drug_pool(
        tuple(dagg2[e] for e in range(8)), hd1, p['g2W1'],
        _row(p['g2b1']), s3, o3,
        p['g2W2'], _row(p['g2b2']), s4, o4, dids)[:B]

    cs, co = _bn_fold(p['cBNg'], p['cBNb'])
    ds_, do = _bn_fold(p['dBg'], p['dBb'])
    f1s, f1o = _bn_fold(p['fB1g'], p['fB1b'])
    f2s, f2o = _bn_fold(p['fB2g'], p['fB2b'])
    out = _tc_head(g_cell, g_drug, (
        p['cL1'], _row(p['cL1b']), cs, co, p['cL2'], _row(p['cL2b']),
        p['dL1'], _row(p['dL1b']), ds_, do, p['dL2'], _row(p['dL2b']),
        p['fW1'], _row(p['fb1']), f1s, f1o,
        p['fW2'], _row(p['fb2']), f2s, f2o,
        p['fW3'], _row(p['fb3'])))
    return out.reshape(B)
